# Initial kernel scaffold; baseline (speedup 1.0000x reference)
import functools

import jax
import jax.numpy as jnp
from jax import lax
from jax.experimental import pallas as pl
from jax.experimental.pallas import tpu as pltpu
from jax.experimental.pallas import tpu_sc as plsc

VOCAB = 100000
HIDDEN = 64
B = 4096
L = 200

_NC = 2
_NS = 16
_NW = _NC * _NS

_IDX_COLS = 100
_ROWS_PER_CHUNK = 4
_CHUNK = _IDX_COLS * _ROWS_PER_CHUNK
_TOTAL = B * L
_PER_W = _TOTAL // _NW
_CHUNKS_PER_W = _PER_W // _CHUNK


def _sc_emb_body(x_hbm, tok_hbm, pos_hbm, out_hbm, pos_v, idx_v, rows_v, sem):
    wid = lax.axis_index("s") * _NC + lax.axis_index("c")
    pltpu.sync_copy(pos_hbm, pos_v)
    row_base = wid * (_PER_W // _IDX_COLS)

    def chunk_body(ch, carry):
        r0 = row_base + ch * _ROWS_PER_CHUNK
        pltpu.sync_copy(x_hbm.at[pl.ds(r0, _ROWS_PER_CHUNK)], idx_v)
        cps = [
            pltpu.async_copy(
                tok_hbm.at[idx_v.at[j]],
                rows_v.at[pl.ds(j * _IDX_COLS, _IDX_COLS)],
                sem,
            )
            for j in range(_ROWS_PER_CHUNK)
        ]
        for cp in cps:
            cp.wait()

        def add_body(l, c2):
            for c in range(HIDDEN // 16):
                sl = pl.ds(c * 16, 16)
                p = pos_v[l, sl]
                rows_v[l, sl] = rows_v[l, sl] + p
                rows_v[l + L, sl] = rows_v[l + L, sl] + p
            return c2

        lax.fori_loop(0, L, add_body, 0)
        pltpu.sync_copy(rows_v, out_hbm.at[pl.ds(r0 * _IDX_COLS, _CHUNK)])
        return carry

    lax.fori_loop(0, _CHUNKS_PER_W, chunk_body, 0)


def _sc_emb(x_rows, token_table, pos_table):
    mesh = plsc.VectorSubcoreMesh(core_axis_name="c", subcore_axis_name="s")
    f = functools.partial(
        pl.kernel,
        mesh=mesh,
        out_type=jax.ShapeDtypeStruct((_TOTAL, HIDDEN), jnp.float32),
        scratch_types=[
            pltpu.VMEM((L, HIDDEN), jnp.float32),
            pltpu.VMEM((_ROWS_PER_CHUNK, _IDX_COLS), jnp.int32),
            pltpu.VMEM((_CHUNK, HIDDEN), jnp.float32),
            pltpu.SemaphoreType.DMA,
        ],
    )(_sc_emb_body)
    return f(x_rows, token_table, pos_table)


_MASK_BB = 128


def _mask_body(x_ref, o_ref):
    m = x_ref[...] > 0
    o_ref[...] = jnp.broadcast_to(m[:, None, :], (_MASK_BB, L, L))


def _tc_mask(x):
    return pl.pallas_call(
        _mask_body,
        grid=(B // _MASK_BB,),
        in_specs=[pl.BlockSpec((_MASK_BB, L), lambda i: (i, 0))],
        out_specs=pl.BlockSpec((_MASK_BB, L, L), lambda i: (i, 0, 0)),
        out_shape=jax.ShapeDtypeStruct((B, L, L), jnp.bool_),
    )(x)


def kernel(x, token_table, pos_table):
    x_rows = x.reshape(_TOTAL // _IDX_COLS, _IDX_COLS)
    emb = _sc_emb(x_rows, token_table, pos_table).reshape(B, L, HIDDEN)
    mask = _tc_mask(x).reshape(B, 1, L, L)
    return emb, mask


# speedup vs baseline: 1.3635x; 1.3635x over previous
import functools

import jax
import jax.numpy as jnp
from jax import lax
from jax.experimental import pallas as pl
from jax.experimental.pallas import tpu as pltpu
from jax.experimental.pallas import tpu_sc as plsc

VOCAB = 100000
HIDDEN = 64
B = 4096
L = 200

_NC = 2
_NS = 16
_NW = _NC * _NS

_IDX_COLS = 100
_ROWS_PER_CHUNK = 4
_CHUNK = _IDX_COLS * _ROWS_PER_CHUNK
_TOTAL = B * L
_PER_W = _TOTAL // _NW
_CHUNKS_PER_W = _PER_W // _CHUNK


def _sc_emb_body(x_hbm, tok_hbm, pos_hbm, out_hbm, pos_v, idx_v, rows_v, sem):
    wid = lax.axis_index("s") * _NC + lax.axis_index("c")
    pltpu.sync_copy(pos_hbm, pos_v)
    row_base = wid * (_PER_W // _IDX_COLS)

    def chunk_body(ch, carry):
        r0 = row_base + ch * _ROWS_PER_CHUNK
        pltpu.sync_copy(x_hbm.at[pl.ds(r0, _ROWS_PER_CHUNK)], idx_v)
        cps = [
            pltpu.async_copy(
                tok_hbm.at[idx_v.at[j]],
                rows_v.at[pl.ds(j * _IDX_COLS, _IDX_COLS)],
                sem,
            )
            for j in range(_ROWS_PER_CHUNK)
        ]
        for cp in cps:
            cp.wait()

        def add_body(l, c2):
            for c in range(HIDDEN // 16):
                sl = pl.ds(c * 16, 16)
                p = pos_v[l, sl]
                rows_v[l, sl] = rows_v[l, sl] + p
                rows_v[l + L, sl] = rows_v[l + L, sl] + p
            return c2

        lax.fori_loop(0, L, add_body, 0)
        pltpu.sync_copy(rows_v, out_hbm.at[pl.ds(r0 * _IDX_COLS, _CHUNK)])
        return carry

    lax.fori_loop(0, _CHUNKS_PER_W, chunk_body, 0)


def _sc_emb(x_rows, token_table, pos_table):
    mesh = plsc.VectorSubcoreMesh(core_axis_name="c", subcore_axis_name="s")
    f = functools.partial(
        pl.kernel,
        mesh=mesh,
        out_type=jax.ShapeDtypeStruct((_TOTAL, HIDDEN), jnp.float32),
        scratch_types=[
            pltpu.VMEM((L, HIDDEN), jnp.float32),
            pltpu.VMEM((_ROWS_PER_CHUNK, _IDX_COLS), jnp.int32),
            pltpu.VMEM((_CHUNK, HIDDEN), jnp.float32),
            pltpu.SemaphoreType.DMA,
        ],
        compiler_params=pltpu.CompilerParams(use_tc_tiling_on_sc=False),
    )(_sc_emb_body)
    return f(x_rows, token_table, pos_table)


_MASK_BB = 128


def _mask_body(x_ref, o_ref):
    m = x_ref[...] > 0
    o_ref[...] = jnp.broadcast_to(m[:, None, :], (_MASK_BB, L, L))


def _tc_mask(x):
    return pl.pallas_call(
        _mask_body,
        grid=(B // _MASK_BB,),
        in_specs=[pl.BlockSpec((_MASK_BB, L), lambda i: (i, 0))],
        out_specs=pl.BlockSpec((_MASK_BB, L, L), lambda i: (i, 0, 0)),
        out_shape=jax.ShapeDtypeStruct((B, L, L), jnp.bool_),
    )(x)


def kernel(x, token_table, pos_table):
    x_rows = x.reshape(_TOTAL // _IDX_COLS, _IDX_COLS)
    emb = _sc_emb(x_rows, token_table, pos_table).reshape(B, L, HIDDEN)
    mask = _tc_mask(x).reshape(B, 1, L, L)
    return emb, mask
